# baseline (speedup 1.0000x reference)
import jax
import jax.numpy as jnp
from jax.experimental import pallas as pl
from jax.experimental.pallas import tpu as pltpu

_N = 1024
_QB = 128
_D = 128
_K = 100000
_KB = 25600
_TB = 256
_NCH = _KB // _TB
_NSTEPS = (_K + _KB - 1) // _KB
_BIG = 3.0e38


def _knn_min_kernel(params_ref, fw_ref, f_ref, mb_ref, out_ref,
                    mbb_ref, m2_ref, acc_ref):
    k = pl.program_id(0)
    ones_row = jnp.ones((1, _D), jnp.float32)

    def prolog(masked):
        for j in range(_NCH):
            mbj = mb_ref[pl.ds(j * _TB, _TB), :]
            m2j = jax.lax.dot_general(
                ones_row, mbj * mbj, (((1,), (1,)), ((), ())),
                preferred_element_type=jnp.float32)
            if masked:
                row = (k * _KB + j * _TB
                       + jax.lax.broadcasted_iota(jnp.int32, (_TB, 1), 0))
                mbj = jnp.where(row < _K, mbj, 0.0)
                col = (k * _KB + j * _TB
                       + jax.lax.broadcasted_iota(jnp.int32, (1, _TB), 1))
                m2j = jnp.where(col < _K, m2j, _BIG)
            mbb_ref[pl.ds(j * _TB, _TB), :] = mbj.astype(jnp.float8_e4m3fn)
            m2_ref[0:1, pl.ds(j * _TB, _TB)] = m2j.astype(jnp.bfloat16)

    @pl.when(k < _NSTEPS - 1)
    def _():
        prolog(False)

    @pl.when(k == _NSTEPS - 1)
    def _():
        prolog(True)

    @pl.when(k == 0)
    def _():
        acc_ref[...] = jnp.full((_N, 128), _BIG, jnp.bfloat16)

    for q in range(_N // _QB):
        fq = fw_ref[:, pl.ds(q * _QB, _QB)]
        pm = None
        for j in range(_NCH):
            mbjb = mbb_ref[pl.ds(j * _TB, _TB), :]
            s = jax.lax.dot_general(
                fq, mbjb, (((0,), (1,)), ((), ())),
                preferred_element_type=jnp.float32)
            part = (s.astype(jnp.bfloat16)
                    + m2_ref[0:1, pl.ds(j * _TB, _TB)])
            for h in range(_TB // 128):
                ph = part[:, h * 128:(h + 1) * 128]
                pm = ph if pm is None else jnp.minimum(pm, ph)
        sl = pl.ds(q * _QB, _QB)
        acc_ref[sl, :] = jnp.minimum(acc_ref[sl, :], pm)

    @pl.when(k == _NSTEPS - 1)
    def _():
        f = f_ref[...]
        f2 = jnp.sum(f * f, axis=1, keepdims=True)
        dmin = jnp.min(acc_ref[...].astype(jnp.float32), axis=1,
                       keepdims=True)
        d2 = jnp.maximum(dmin + f2, 1e-12)
        d = jnp.sqrt(d2)
        mn = params_ref[0, 0]
        mx = params_ref[0, 1]
        out_ref[...] = (d - mn) / (mx - mn)


def kernel(features, memory_bank, min_val, max_val):
    params = jnp.stack([min_val, max_val]).reshape(1, 2)
    fw = (features * -2.0).astype(jnp.float8_e4m3fn).T
    out = pl.pallas_call(
        _knn_min_kernel,
        grid=(_NSTEPS,),
        in_specs=[
            pl.BlockSpec(memory_space=pltpu.SMEM),
            pl.BlockSpec((_D, _N), lambda k: (0, 0)),
            pl.BlockSpec((_N, _D), lambda k: (0, 0)),
            pl.BlockSpec((_KB, _D), lambda k: (k, 0)),
        ],
        out_specs=pl.BlockSpec((_N, 1), lambda k: (0, 0)),
        out_shape=jax.ShapeDtypeStruct((_N, 1), jnp.float32),
        scratch_shapes=[
            pltpu.VMEM((_KB, _D), jnp.float8_e4m3fn),
            pltpu.VMEM((8, _KB), jnp.bfloat16),
            pltpu.VMEM((_N, 128), jnp.bfloat16),
        ],
        compiler_params=pltpu.CompilerParams(
            dimension_semantics=("arbitrary",)),
    )(params, fw, features, memory_bank)
    return out.reshape(_N)


# speedup vs baseline: 1.0094x; 1.0094x over previous
import jax
import jax.numpy as jnp
from jax.experimental import pallas as pl
from jax.experimental.pallas import tpu as pltpu

_N = 1024
_QB = 256
_D = 128
_K = 100000
_KB = 12800
_TB = 256
_NCH = _KB // _TB
_NSTEPS = (_K + _KB - 1) // _KB
_BIG = 3.0e38


def _knn_min_kernel(params_ref, fw_ref, f_ref, mb_ref, out_ref,
                    mbb_ref, m2_ref, acc_ref):
    k = pl.program_id(0)
    ones_row = jnp.ones((1, _D), jnp.float32)

    def prolog(masked):
        for j in range(_NCH):
            mbj = mb_ref[pl.ds(j * _TB, _TB), :]
            m2j = jax.lax.dot_general(
                ones_row, mbj * mbj, (((1,), (1,)), ((), ())),
                preferred_element_type=jnp.float32)
            if masked:
                row = (k * _KB + j * _TB
                       + jax.lax.broadcasted_iota(jnp.int32, (_TB, 1), 0))
                mbj = jnp.where(row < _K, mbj, 0.0)
                col = (k * _KB + j * _TB
                       + jax.lax.broadcasted_iota(jnp.int32, (1, _TB), 1))
                m2j = jnp.where(col < _K, m2j, _BIG)
            mbb_ref[pl.ds(j * _TB, _TB), :] = mbj.astype(jnp.float8_e4m3fn)
            m2_ref[0:1, pl.ds(j * _TB, _TB)] = m2j.astype(jnp.bfloat16)

    @pl.when(k < _NSTEPS - 1)
    def _():
        prolog(False)

    @pl.when(k == _NSTEPS - 1)
    def _():
        prolog(True)

    @pl.when(k == 0)
    def _():
        acc_ref[...] = jnp.full((_N, 128), _BIG, jnp.bfloat16)

    for q in range(_N // _QB):
        fq = fw_ref[:, pl.ds(q * _QB, _QB)]
        pm = None
        for j in range(_NCH):
            mbjb = mbb_ref[pl.ds(j * _TB, _TB), :]
            s = jax.lax.dot_general(
                fq, mbjb, (((0,), (1,)), ((), ())),
                preferred_element_type=jnp.float32)
            part = (s.astype(jnp.bfloat16)
                    + m2_ref[0:1, pl.ds(j * _TB, _TB)])
            for h in range(_TB // 128):
                ph = part[:, h * 128:(h + 1) * 128]
                pm = ph if pm is None else jnp.minimum(pm, ph)
        sl = pl.ds(q * _QB, _QB)
        acc_ref[sl, :] = jnp.minimum(acc_ref[sl, :], pm)

    @pl.when(k == _NSTEPS - 1)
    def _():
        f = f_ref[...]
        f2 = jnp.sum(f * f, axis=1, keepdims=True)
        dmin = jnp.min(acc_ref[...].astype(jnp.float32), axis=1,
                       keepdims=True)
        d2 = jnp.maximum(dmin + f2, 1e-12)
        d = jnp.sqrt(d2)
        mn = params_ref[0, 0]
        mx = params_ref[0, 1]
        out_ref[...] = (d - mn) / (mx - mn)


def kernel(features, memory_bank, min_val, max_val):
    params = jnp.stack([min_val, max_val]).reshape(1, 2)
    fw = (features * -2.0).astype(jnp.float8_e4m3fn).T
    out = pl.pallas_call(
        _knn_min_kernel,
        grid=(_NSTEPS,),
        in_specs=[
            pl.BlockSpec(memory_space=pltpu.SMEM),
            pl.BlockSpec((_D, _N), lambda k: (0, 0)),
            pl.BlockSpec((_N, _D), lambda k: (0, 0)),
            pl.BlockSpec((_KB, _D), lambda k: (k, 0)),
        ],
        out_specs=pl.BlockSpec((_N, 1), lambda k: (0, 0)),
        out_shape=jax.ShapeDtypeStruct((_N, 1), jnp.float32),
        scratch_shapes=[
            pltpu.VMEM((_KB, _D), jnp.float8_e4m3fn),
            pltpu.VMEM((8, _KB), jnp.bfloat16),
            pltpu.VMEM((_N, 128), jnp.bfloat16),
        ],
        compiler_params=pltpu.CompilerParams(
            dimension_semantics=("arbitrary",)),
    )(params, fw, features, memory_bank)
    return out.reshape(_N)
